# baseline (speedup 1.0000x reference)
import functools

import jax
import jax.numpy as jnp
from jax.experimental import pallas as pl
from jax.experimental.pallas import tpu as pltpu
from jax.experimental.pallas import tpu_sc as plsc

_B = 16384
_D = 32
_F = 26
_SPARSE_V = 100000
_DIN = (_F + 2) * _D
_H = 2 * _D
_W = 128
_BB = 512
_NJ = _F + 2


_RB = 2000


def _detile_body(x_ref, o_ref):
    c = pl.program_id(1)
    x = x_ref[...]
    for cc in range(4):
        @pl.when(c == cc)
        def _():
            o_ref[:, 32 * cc:32 * cc + 32] = x


def _detile(table):
    v = table.shape[0]
    q = v // 4
    nb = q // _RB
    return pl.pallas_call(
        _detile_body,
        grid=(nb, 4),
        in_specs=[pl.BlockSpec((_RB, _D), lambda i, c, n=nb: (n * c + i, 0))],
        out_specs=pl.BlockSpec((_RB, 128), lambda i, c: (i, 0)),
        out_shape=jax.ShapeDtypeStruct((q, 128), jnp.float32),
    )(table)


def _gather_pipeline(table_hbm, idx_hbm, idx_row, out_hbm):
    def body(i_vmem, o_vmem):
        pltpu.sync_copy(table_hbm.at[i_vmem.at[0]], o_vmem)

    pltpu.emit_pipeline(
        body,
        grid=(_B // _W,),
        in_specs=[pl.BlockSpec((1, _W), lambda i, r=idx_row: (r, i))],
        out_specs=[pl.BlockSpec((_W, 128), lambda i: (i, 0))],
        core_axis_name=("c", "s"),
        dimension_semantics=(pltpu.PARALLEL,),
    )(idx_hbm, out_hbm)


@functools.cache
def _sc_gather_kernel(n_idx_rows):
    mesh = plsc.VectorSubcoreMesh(core_axis_name="c", subcore_axis_name="s")
    emb = jax.ShapeDtypeStruct((_B, 128), jnp.float32)

    @functools.partial(
        pl.kernel,
        out_type=(emb,) * n_idx_rows,
        mesh=mesh,
    )
    def sc_gather(tbl_hbm, idx_hbm, *outs):
        for f in range(n_idx_rows):
            _gather_pipeline(tbl_hbm, idx_hbm, f, outs[f])

    return sc_gather


def _dense_body(*refs):
    x_refs = refs[:_NJ]
    (sel_ref, wWb_ref, wb_ref, w0b_ref, b0_ref, w1_ref, b1_ref,
     w2_ref, b2_ref, w3_ref, b3_ref, tw_ref, tb_ref, o_ref) = refs[_NJ:]
    dot = lambda a, b: jax.lax.dot_general(
        a, b, (((1,), (0,)), ((), ())), preferred_element_type=jnp.float32)
    w0b = w0b_ref[...]
    wWb = wWb_ref[...]
    lane_q = jax.lax.broadcasted_iota(jnp.int32, (_BB, 128), 1) // 32
    hacc = None
    wacc = None
    for j in range(_NJ):
        x = x_refs[j][...]
        sel = sel_ref[:, j:j + 1]
        e = jnp.where(lane_q == sel, x, 0.0)
        hj = dot(e, w0b[128 * j:128 * j + 128])
        wj = dot(e, wWb[128 * j:128 * j + 128])
        hacc = hj if hacc is None else hacc + hj
        wacc = wj if wacc is None else wacc + wj
    h = jax.nn.relu(hacc + b0_ref[...])
    h = jax.nn.relu(dot(h, w1_ref[...]) + b1_ref[...])
    h = jax.nn.relu(dot(h, w2_ref[...]) + b2_ref[...])
    deep = dot(h, w3_ref[...]) + b3_ref[...]
    wide = wacc + wb_ref[...]
    tw = tw_ref[...]
    logit = (jnp.sum(wide * tw[:, 0:_D], axis=1, keepdims=True)
             + jnp.sum(deep * tw[:, _D:], axis=1, keepdims=True)
             + tb_ref[...])
    o_ref[...] = jax.nn.sigmoid(logit)


def _dense_forward(xs, sel, wide_W, wide_b, dW0, db0, dW1, db1,
                   dW2, db2, dW3, db3, tW, tb):
    row = lambda i: (i, 0)
    full = lambda i: (0, 0)
    w0b = jnp.tile(dW0.reshape(_NJ, 1, _D, _H), (1, 4, 1, 1)).reshape(_NJ * 128, _H)
    wWb = jnp.tile(wide_W.reshape(_NJ, 1, _D, _D), (1, 4, 1, 1)).reshape(_NJ * 128, _D)
    return pl.pallas_call(
        _dense_body,
        grid=(_B // _BB,),
        in_specs=[pl.BlockSpec((_BB, 128), row)] * _NJ + [
            pl.BlockSpec((_BB, _NJ), row),
            pl.BlockSpec((_NJ * 128, _D), full),
            pl.BlockSpec((1, _D), full),
            pl.BlockSpec((_NJ * 128, _H), full),
            pl.BlockSpec((1, _H), full),
            pl.BlockSpec((_H, _H), full),
            pl.BlockSpec((1, _H), full),
            pl.BlockSpec((_H, _H), full),
            pl.BlockSpec((1, _H), full),
            pl.BlockSpec((_H, _D), full),
            pl.BlockSpec((1, _D), full),
            pl.BlockSpec((1, 2 * _D), full),
            pl.BlockSpec((1, 1), full),
        ],
        out_specs=pl.BlockSpec((_BB, 1), row),
        out_shape=jax.ShapeDtypeStruct((_B, 1), jnp.float32),
    )(*xs, sel, wWb, wide_b.reshape(1, _D), w0b, db0.reshape(1, _H),
      dW1, db1.reshape(1, _H), dW2, db2.reshape(1, _H), dW3,
      db3.reshape(1, _D), tW.reshape(1, 2 * _D), tb.reshape(1, 1))


def kernel(user_ids, item_ids, sparse_features, user_table, item_table,
           sparse_tables, wide_W, wide_b, dW0, db0, dW1, db1, dW2, db2,
           dW3, db3, tW, tb):
    s4 = _detile(sparse_tables.reshape(_F * _SPARSE_V, _D))
    u4 = _detile(user_table)
    i4 = _detile(item_table)
    qu = (_SPARSE_V * 10) // 4
    qs = (_F * _SPARSE_V) // 4
    ui = (user_ids % qu).reshape(1, _B)
    ii = (item_ids % qu).reshape(1, _B)
    sf_t = sparse_features.T
    offs_col = (jnp.arange(_F, dtype=jnp.int32) * _SPARSE_V)[:, None]
    si = (sf_t + offs_col) % qs
    offs_row = (jnp.arange(_F, dtype=jnp.int32) * _SPARSE_V)[None, :]
    sel = jnp.concatenate([(user_ids // qu)[:, None], (item_ids // qu)[:, None],
                           (sparse_features + offs_row) // qs], axis=1)
    xs_s = _sc_gather_kernel(_F)(s4, si)
    (xs_u,) = _sc_gather_kernel(1)(u4, ui)
    (xs_i,) = _sc_gather_kernel(1)(i4, ii)
    xs = (xs_u, xs_i) + tuple(xs_s)
    return _dense_forward(xs, sel, wide_W, wide_b, dW0, db0,
                          dW1, db1, dW2, db2, dW3, db3, tW, tb)


# speedup vs baseline: 1.8207x; 1.8207x over previous
import functools

import jax
import jax.numpy as jnp
from jax.experimental import pallas as pl
from jax.experimental.pallas import tpu as pltpu
from jax.experimental.pallas import tpu_sc as plsc

_B = 16384
_D = 32
_F = 26
_SPARSE_V = 100000
_DIN = (_F + 2) * _D
_H = 2 * _D
_W = 128
_BB = 512
_NJ = _F + 2


_RB = 2000


def _detile_body(x_ref, o_ref):
    for c in range(4):
        o_ref[:, 32 * c:32 * c + 32] = x_ref[c]


def _detile(table):
    v = table.shape[0]
    q = v // 4
    return pl.pallas_call(
        _detile_body,
        grid=(q // _RB,),
        in_specs=[pl.BlockSpec((4, _RB, _D), lambda i: (0, i, 0))],
        out_specs=pl.BlockSpec((_RB, 128), lambda i: (i, 0)),
        out_shape=jax.ShapeDtypeStruct((q, 128), jnp.float32),
    )(table.reshape(4, q, _D))


def _gather_pipeline(table_hbm, idx_hbm, idx_row, out_hbm):
    def body(i_vmem, o_vmem):
        pltpu.sync_copy(table_hbm.at[i_vmem.at[0]], o_vmem)

    pltpu.emit_pipeline(
        body,
        grid=(_B // _W,),
        in_specs=[pl.BlockSpec((1, _W), lambda i, r=idx_row: (r, i))],
        out_specs=[pl.BlockSpec((_W, 128), lambda i: (i, 0))],
        core_axis_name=("c", "s"),
        dimension_semantics=(pltpu.PARALLEL,),
    )(idx_hbm, out_hbm)


@functools.cache
def _sc_gather_kernel(n_idx_rows):
    mesh = plsc.VectorSubcoreMesh(core_axis_name="c", subcore_axis_name="s")
    emb = jax.ShapeDtypeStruct((_B, 128), jnp.float32)

    @functools.partial(
        pl.kernel,
        out_type=(emb,) * n_idx_rows,
        mesh=mesh,
    )
    def sc_gather(tbl_hbm, idx_hbm, *outs):
        for f in range(n_idx_rows):
            _gather_pipeline(tbl_hbm, idx_hbm, f, outs[f])

    return sc_gather


def _dense_body(*refs):
    x_refs = refs[:_NJ]
    (sel_ref, wWb_ref, wb_ref, w0b_ref, b0_ref, w1_ref, b1_ref,
     w2_ref, b2_ref, w3_ref, b3_ref, tw_ref, tb_ref, o_ref) = refs[_NJ:]
    dot = lambda a, b: jax.lax.dot_general(
        a, b, (((1,), (0,)), ((), ())), preferred_element_type=jnp.float32)
    w0b = w0b_ref[...]
    wWb = wWb_ref[...]
    lane_q = jax.lax.broadcasted_iota(jnp.int32, (_BB, 128), 1) // 32
    hacc = None
    wacc = None
    for j in range(_NJ):
        x = x_refs[j][...]
        sel = sel_ref[:, j:j + 1]
        e = jnp.where(lane_q == sel, x, 0.0)
        hj = dot(e, w0b[128 * j:128 * j + 128])
        wj = dot(e, wWb[128 * j:128 * j + 128])
        hacc = hj if hacc is None else hacc + hj
        wacc = wj if wacc is None else wacc + wj
    h = jax.nn.relu(hacc + b0_ref[...])
    h = jax.nn.relu(dot(h, w1_ref[...]) + b1_ref[...])
    h = jax.nn.relu(dot(h, w2_ref[...]) + b2_ref[...])
    deep = dot(h, w3_ref[...]) + b3_ref[...]
    wide = wacc + wb_ref[...]
    tw = tw_ref[...]
    logit = (jnp.sum(wide * tw[:, 0:_D], axis=1, keepdims=True)
             + jnp.sum(deep * tw[:, _D:], axis=1, keepdims=True)
             + tb_ref[...])
    o_ref[...] = jax.nn.sigmoid(logit)


def _dense_forward(xs, sel, wide_W, wide_b, dW0, db0, dW1, db1,
                   dW2, db2, dW3, db3, tW, tb):
    row = lambda i: (i, 0)
    full = lambda i: (0, 0)
    w0b = jnp.tile(dW0.reshape(_NJ, 1, _D, _H), (1, 4, 1, 1)).reshape(_NJ * 128, _H)
    wWb = jnp.tile(wide_W.reshape(_NJ, 1, _D, _D), (1, 4, 1, 1)).reshape(_NJ * 128, _D)
    return pl.pallas_call(
        _dense_body,
        grid=(_B // _BB,),
        in_specs=[pl.BlockSpec((_BB, 128), row)] * _NJ + [
            pl.BlockSpec((_BB, _NJ), row),
            pl.BlockSpec((_NJ * 128, _D), full),
            pl.BlockSpec((1, _D), full),
            pl.BlockSpec((_NJ * 128, _H), full),
            pl.BlockSpec((1, _H), full),
            pl.BlockSpec((_H, _H), full),
            pl.BlockSpec((1, _H), full),
            pl.BlockSpec((_H, _H), full),
            pl.BlockSpec((1, _H), full),
            pl.BlockSpec((_H, _D), full),
            pl.BlockSpec((1, _D), full),
            pl.BlockSpec((1, 2 * _D), full),
            pl.BlockSpec((1, 1), full),
        ],
        out_specs=pl.BlockSpec((_BB, 1), row),
        out_shape=jax.ShapeDtypeStruct((_B, 1), jnp.float32),
    )(*xs, sel, wWb, wide_b.reshape(1, _D), w0b, db0.reshape(1, _H),
      dW1, db1.reshape(1, _H), dW2, db2.reshape(1, _H), dW3,
      db3.reshape(1, _D), tW.reshape(1, 2 * _D), tb.reshape(1, 1))


def kernel(user_ids, item_ids, sparse_features, user_table, item_table,
           sparse_tables, wide_W, wide_b, dW0, db0, dW1, db1, dW2, db2,
           dW3, db3, tW, tb):
    s4 = _detile(sparse_tables.reshape(_F * _SPARSE_V, _D))
    u4 = _detile(user_table)
    i4 = _detile(item_table)
    qu = (_SPARSE_V * 10) // 4
    qs = (_F * _SPARSE_V) // 4
    ui = (user_ids % qu).reshape(1, _B)
    ii = (item_ids % qu).reshape(1, _B)
    sf_t = sparse_features.T
    offs_col = (jnp.arange(_F, dtype=jnp.int32) * _SPARSE_V)[:, None]
    si = (sf_t + offs_col) % qs
    offs_row = (jnp.arange(_F, dtype=jnp.int32) * _SPARSE_V)[None, :]
    sel = jnp.concatenate([(user_ids // qu)[:, None], (item_ids // qu)[:, None],
                           (sparse_features + offs_row) // qs], axis=1)
    xs_s = _sc_gather_kernel(_F)(s4, si)
    (xs_u,) = _sc_gather_kernel(1)(u4, ui)
    (xs_i,) = _sc_gather_kernel(1)(i4, ii)
    xs = (xs_u, xs_i) + tuple(xs_s)
    return _dense_forward(xs, sel, wide_W, wide_b, dW0, db0,
                          dW1, db1, dW2, db2, dW3, db3, tW, tb)


# speedup vs baseline: 1.9268x; 1.0583x over previous
import functools

import jax
import jax.numpy as jnp
from jax.experimental import pallas as pl
from jax.experimental.pallas import tpu as pltpu
from jax.experimental.pallas import tpu_sc as plsc

_B = 16384
_D = 32
_F = 26
_SPARSE_V = 100000
_DIN = (_F + 2) * _D
_H = 2 * _D
_W = 128
_BB = 512
_NJ = _F + 2


_RB = 10000


def _detile_body(x_ref, o_ref):
    for c in range(4):
        o_ref[:, 32 * c:32 * c + 32] = x_ref[c]


def _detile(table):
    v = table.shape[0]
    q = v // 4
    return pl.pallas_call(
        _detile_body,
        grid=(q // _RB,),
        in_specs=[pl.BlockSpec((4, _RB, _D), lambda i: (0, i, 0))],
        out_specs=pl.BlockSpec((_RB, 128), lambda i: (i, 0)),
        out_shape=jax.ShapeDtypeStruct((q, 128), jnp.float32),
    )(table.reshape(4, q, _D))


def _gather_pipeline(table_hbm, idx_hbm, idx_row, out_hbm):
    def body(i_vmem, o_vmem):
        pltpu.sync_copy(table_hbm.at[i_vmem.at[0]], o_vmem)

    pltpu.emit_pipeline(
        body,
        grid=(_B // _W,),
        in_specs=[pl.BlockSpec((1, _W), lambda i, r=idx_row: (r, i))],
        out_specs=[pl.BlockSpec((_W, 128), lambda i: (i, 0))],
        core_axis_name=("c", "s"),
        dimension_semantics=(pltpu.PARALLEL,),
    )(idx_hbm, out_hbm)


@functools.cache
def _sc_gather_kernel(n_idx_rows):
    mesh = plsc.VectorSubcoreMesh(core_axis_name="c", subcore_axis_name="s")
    emb = jax.ShapeDtypeStruct((_B, 128), jnp.float32)

    @functools.partial(
        pl.kernel,
        out_type=(emb,) * n_idx_rows,
        mesh=mesh,
    )
    def sc_gather(tbl_hbm, idx_hbm, *outs):
        for f in range(n_idx_rows):
            _gather_pipeline(tbl_hbm, idx_hbm, f, outs[f])

    return sc_gather


def _dense_body(*refs):
    x_refs = refs[:_NJ]
    (sel_ref, wWb_ref, wb_ref, w0b_ref, b0_ref, w1_ref, b1_ref,
     w2_ref, b2_ref, w3_ref, b3_ref, tw_ref, tb_ref, o_ref) = refs[_NJ:]
    dot = lambda a, b: jax.lax.dot_general(
        a, b, (((1,), (0,)), ((), ())), preferred_element_type=jnp.float32)
    w0b = w0b_ref[...]
    wWb = wWb_ref[...]
    lane_q = jax.lax.broadcasted_iota(jnp.int32, (_BB, 128), 1) // 32
    hacc = None
    wacc = None
    for j in range(_NJ):
        x = x_refs[j][...]
        sel = sel_ref[:, j:j + 1]
        e = jnp.where(lane_q == sel, x, 0.0)
        hj = dot(e, w0b[128 * j:128 * j + 128])
        wj = dot(e, wWb[128 * j:128 * j + 128])
        hacc = hj if hacc is None else hacc + hj
        wacc = wj if wacc is None else wacc + wj
    h = jax.nn.relu(hacc + b0_ref[...])
    h = jax.nn.relu(dot(h, w1_ref[...]) + b1_ref[...])
    h = jax.nn.relu(dot(h, w2_ref[...]) + b2_ref[...])
    deep = dot(h, w3_ref[...]) + b3_ref[...]
    wide = wacc + wb_ref[...]
    tw = tw_ref[...]
    logit = (jnp.sum(wide * tw[:, 0:_D], axis=1, keepdims=True)
             + jnp.sum(deep * tw[:, _D:], axis=1, keepdims=True)
             + tb_ref[...])
    o_ref[...] = jax.nn.sigmoid(logit)


def _dense_forward(xs, sel, wide_W, wide_b, dW0, db0, dW1, db1,
                   dW2, db2, dW3, db3, tW, tb):
    row = lambda i: (i, 0)
    full = lambda i: (0, 0)
    w0b = jnp.tile(dW0.reshape(_NJ, 1, _D, _H), (1, 4, 1, 1)).reshape(_NJ * 128, _H)
    wWb = jnp.tile(wide_W.reshape(_NJ, 1, _D, _D), (1, 4, 1, 1)).reshape(_NJ * 128, _D)
    return pl.pallas_call(
        _dense_body,
        grid=(_B // _BB,),
        in_specs=[pl.BlockSpec((_BB, 128), row)] * _NJ + [
            pl.BlockSpec((_BB, _NJ), row),
            pl.BlockSpec((_NJ * 128, _D), full),
            pl.BlockSpec((1, _D), full),
            pl.BlockSpec((_NJ * 128, _H), full),
            pl.BlockSpec((1, _H), full),
            pl.BlockSpec((_H, _H), full),
            pl.BlockSpec((1, _H), full),
            pl.BlockSpec((_H, _H), full),
            pl.BlockSpec((1, _H), full),
            pl.BlockSpec((_H, _D), full),
            pl.BlockSpec((1, _D), full),
            pl.BlockSpec((1, 2 * _D), full),
            pl.BlockSpec((1, 1), full),
        ],
        out_specs=pl.BlockSpec((_BB, 1), row),
        out_shape=jax.ShapeDtypeStruct((_B, 1), jnp.float32),
    )(*xs, sel, wWb, wide_b.reshape(1, _D), w0b, db0.reshape(1, _H),
      dW1, db1.reshape(1, _H), dW2, db2.reshape(1, _H), dW3,
      db3.reshape(1, _D), tW.reshape(1, 2 * _D), tb.reshape(1, 1))


def kernel(user_ids, item_ids, sparse_features, user_table, item_table,
           sparse_tables, wide_W, wide_b, dW0, db0, dW1, db1, dW2, db2,
           dW3, db3, tW, tb):
    s4 = _detile(sparse_tables.reshape(_F * _SPARSE_V, _D))
    u4 = _detile(user_table)
    i4 = _detile(item_table)
    qu = (_SPARSE_V * 10) // 4
    qs = (_F * _SPARSE_V) // 4
    ui = (user_ids % qu).reshape(1, _B)
    ii = (item_ids % qu).reshape(1, _B)
    sf_t = sparse_features.T
    offs_col = (jnp.arange(_F, dtype=jnp.int32) * _SPARSE_V)[:, None]
    si = (sf_t + offs_col) % qs
    offs_row = (jnp.arange(_F, dtype=jnp.int32) * _SPARSE_V)[None, :]
    sel = jnp.concatenate([(user_ids // qu)[:, None], (item_ids // qu)[:, None],
                           (sparse_features + offs_row) // qs], axis=1)
    xs_s = _sc_gather_kernel(_F)(s4, si)
    (xs_u,) = _sc_gather_kernel(1)(u4, ui)
    (xs_i,) = _sc_gather_kernel(1)(i4, ii)
    xs = (xs_u, xs_i) + tuple(xs_s)
    return _dense_forward(xs, sel, wide_W, wide_b, dW0, db0,
                          dW1, db1, dW2, db2, dW3, db3, tW, tb)
